# Initial kernel scaffold; baseline (speedup 1.0000x reference)
import jax
import jax.numpy as jnp
from jax.experimental import pallas as pl


def kernel(htilde, stilde):
    raise NotImplementedError("write your pallas kernel here")


# speedup vs baseline: 1.7534x; 1.7534x over previous
import functools

import jax
import jax.numpy as jnp
from jax import lax
from jax.experimental import pallas as pl
from jax.experimental.pallas import tpu as pltpu
from jax.experimental.pallas import tpu_sc as plsc

NFREQ = 8193
PAD = 8208
NVEC = PAD // 16
NROWS = 512
NWORKERS = 32
ROWS_PER_W = NROWS // NWORKERS
SCALE = jnp.float32(0.5)


def _rsqrt_vec(t):
    i = plsc.bitcast(t, jnp.int32)
    r = plsc.bitcast(jnp.int32(0x5F3759DF) - (i >> 1), jnp.float32)
    for _ in range(3):
        r = r * (jnp.float32(1.5) - jnp.float32(0.5) * t * r * r)
    return r


_MESH = plsc.VectorSubcoreMesh(core_axis_name="c", subcore_axis_name="s")


@functools.partial(
    pl.kernel,
    mesh=_MESH,
    compiler_params=pltpu.CompilerParams(
        needs_layout_passes=False, use_tc_tiling_on_sc=False
    ),
    out_type=(
        jax.ShapeDtypeStruct((NROWS,), jnp.float32),
        jax.ShapeDtypeStruct((NROWS,), jnp.float32),
    ),
    scratch_types=[
        pltpu.VMEM((PAD,), jnp.float32),
        pltpu.VMEM((PAD,), jnp.float32),
        pltpu.VMEM((16,), jnp.int32),
        pltpu.VMEM((16,), jnp.float32),
        pltpu.VMEM((16,), jnp.float32),
    ],
)
def _chisq_sc(h_hbm, s_hbm, snr_hbm, chisq_hbm, hbuf, sbuf, ebuf, snrbuf, chibuf):
    wid = lax.axis_index("s") * 2 + lax.axis_index("c")
    row0 = wid * ROWS_PER_W
    lanes = lax.iota(jnp.int32, 16)
    fzero = jnp.zeros((16,), jnp.float32)

    def row_body(i, carry):
        snr_res, chi_res = carry
        row = row0 + i
        pltpu.sync_copy(h_hbm.at[row], hbuf.at[pl.ds(0, NFREQ)])
        pltpu.sync_copy(s_hbm.at[row], sbuf.at[pl.ds(0, NFREQ)])
        tail_h = hbuf[pl.ds(NFREQ - 1, 16)]
        hbuf[pl.ds(NFREQ - 1, 16)] = jnp.where(lanes == 0, tail_h, fzero)
        tail_s = sbuf[pl.ds(NFREQ - 1, 16)]
        sbuf[pl.ds(NFREQ - 1, 16)] = jnp.where(lanes == 0, tail_s, fzero)

        def vec_body(j, c):
            cx, cy = c
            hv = hbuf[pl.ds(j * 16, 16)]
            sv = sbuf[pl.ds(j * 16, 16)]
            x = hv * hv * SCALE
            y = jnp.abs(hv * sv) * SCALE
            xi = plsc.cumsum(x) + cx
            yi = plsc.cumsum(y) + cy
            hbuf[pl.ds(j * 16, 16)] = xi
            sbuf[pl.ds(j * 16, 16)] = yi
            return jnp.max(xi), jnp.max(yi)

        total, sum_y = lax.fori_loop(
            0, NVEC, vec_body, (jnp.float32(0.0), jnp.float32(0.0))
        )

        t_vec = lanes.astype(jnp.float32) * (total * jnp.float32(1.0 / 16.0))

        def bs_body(_, c):
            lo, hi = c
            mid = (lo + hi) >> 1
            vals = plsc.load_gather(hbuf, [mid])
            pred = vals <= t_vec
            return jnp.where(pred, mid + 1, lo), jnp.where(pred, hi, mid)

        lo, _ = lax.fori_loop(
            0, 14, bs_body,
            (jnp.zeros((16,), jnp.int32), jnp.full((16,), PAD, jnp.int32)),
        )
        e = jnp.minimum(lo, NFREQ - 1)
        left = plsc.load_gather(sbuf, [e])
        ebuf[...] = e
        ridx = plsc.load_gather(ebuf, [jnp.minimum(lanes + 1, 15)])
        ridx = jnp.where(lanes == 15, NFREQ - 1, ridx)
        right = plsc.load_gather(sbuf, [ridx])

        rs = _rsqrt_vec(jnp.zeros((16,), jnp.float32) + total)
        spb = (right - left) * rs
        snr_splat = sum_y * rs
        d = spb - snr_splat * jnp.float32(1.0 / 16.0)
        chi = jnp.sum(d * d) * jnp.float32(16.0 / 15.0)
        snr_res = jnp.where(lanes == i, snr_splat, snr_res)
        chi_res = jnp.where(lanes == i, chi, chi_res)
        return snr_res, chi_res

    snr_res, chi_res = lax.fori_loop(0, ROWS_PER_W, row_body, (fzero, fzero))
    snrbuf[...] = snr_res
    chibuf[...] = chi_res
    pltpu.sync_copy(snrbuf, snr_hbm.at[pl.ds(row0, ROWS_PER_W)])
    pltpu.sync_copy(chibuf, chisq_hbm.at[pl.ds(row0, ROWS_PER_W)])


def kernel(htilde, stilde):
    b, c, f = htilde.shape
    snr, chisq = _chisq_sc(htilde.reshape(b * c, f), stilde.reshape(b * c, f))
    return snr.reshape(b, c), chisq.reshape(b, c)


# speedup vs baseline: 2.5649x; 1.4628x over previous
import functools

import jax
import jax.numpy as jnp
from jax import lax
from jax.experimental import pallas as pl
from jax.experimental.pallas import tpu as pltpu
from jax.experimental.pallas import tpu_sc as plsc

NFREQ = 8193
PAD = 8320
NVEC = PAD // 16
UNROLL = 8
NGROUP = NVEC // UNROLL
NROWS = 512
NWORKERS = 32
ROWS_PER_W = NROWS // NWORKERS
SCALE = jnp.float32(0.5)


def _rsqrt_vec(t):
    i = plsc.bitcast(t, jnp.int32)
    r = plsc.bitcast(jnp.int32(0x5F3759DF) - (i >> 1), jnp.float32)
    for _ in range(3):
        r = r * (jnp.float32(1.5) - jnp.float32(0.5) * t * r * r)
    return r


_MESH = plsc.VectorSubcoreMesh(core_axis_name="c", subcore_axis_name="s")


@functools.partial(
    pl.kernel,
    mesh=_MESH,
    compiler_params=pltpu.CompilerParams(
        needs_layout_passes=False, use_tc_tiling_on_sc=False
    ),
    out_type=(
        jax.ShapeDtypeStruct((NROWS,), jnp.float32),
        jax.ShapeDtypeStruct((NROWS,), jnp.float32),
    ),
    scratch_types=[
        pltpu.VMEM((PAD,), jnp.float32),
        pltpu.VMEM((PAD,), jnp.float32),
        pltpu.VMEM((PAD,), jnp.float32),
        pltpu.VMEM((PAD,), jnp.float32),
        pltpu.VMEM((PAD,), jnp.float32),
        pltpu.VMEM((PAD,), jnp.float32),
        pltpu.VMEM((16,), jnp.int32),
        pltpu.VMEM((16,), jnp.float32),
        pltpu.VMEM((16,), jnp.float32),
        pltpu.SemaphoreType.DMA,
        pltpu.SemaphoreType.DMA,
        pltpu.SemaphoreType.DMA,
        pltpu.SemaphoreType.DMA,
    ],
)
def _chisq_sc(h_hbm, s_hbm, snr_hbm, chisq_hbm, h_a, h_b, s_a, s_b, xbuf,
              ybuf, ebuf, snrbuf, chibuf, sem_ha, sem_sa, sem_hb, sem_sb):
    wid = lax.axis_index("s") * 2 + lax.axis_index("c")
    row0 = wid * ROWS_PER_W
    lanes = lax.iota(jnp.int32, 16)
    fzero = jnp.zeros((16,), jnp.float32)

    for buf in (h_a, h_b, s_a, s_b):
        for k in range(8):
            buf[pl.ds(NFREQ - 1 + 16 * k, 16)] = fzero

    def start_row(row, hbuf, sbuf, sem_h, sem_s):
        pltpu.make_async_copy(h_hbm.at[row], hbuf.at[pl.ds(0, NFREQ)], sem_h).start()
        pltpu.make_async_copy(s_hbm.at[row], sbuf.at[pl.ds(0, NFREQ)], sem_s).start()

    def wait_row(row, hbuf, sbuf, sem_h, sem_s):
        pltpu.make_async_copy(h_hbm.at[row], hbuf.at[pl.ds(0, NFREQ)], sem_h).wait()
        pltpu.make_async_copy(s_hbm.at[row], sbuf.at[pl.ds(0, NFREQ)], sem_s).wait()

    def process_row(hbuf, sbuf):
        def grp(gi, c):
            off_x, off_y = c
            base = gi * (UNROLL * 16)
            parts = []
            for u in range(UNROLL):
                o = base + u * 16
                hv = hbuf[pl.ds(o, 16)]
                sv = sbuf[pl.ds(o, 16)]
                x = hv * hv * SCALE
                y = jnp.abs(hv * sv) * SCALE
                xc = plsc.cumsum(x)
                yc = plsc.cumsum(y)
                parts.append((o, xc, yc, jnp.max(xc), jnp.max(yc)))
            for o, xc, yc, sx, sy in parts:
                xbuf[pl.ds(o, 16)] = xc + off_x
                ybuf[pl.ds(o, 16)] = yc + off_y
                off_x = off_x + sx
                off_y = off_y + sy
            return off_x, off_y

        total, sum_y = lax.fori_loop(
            0, NGROUP, grp, (jnp.float32(0.0), jnp.float32(0.0))
        )

        t_vec = lanes.astype(jnp.float32) * (total * jnp.float32(1.0 / 16.0))

        def bs_body(_, c):
            lo, hi = c
            mid = (lo + hi) >> 1
            vals = plsc.load_gather(xbuf, [mid])
            pred = vals <= t_vec
            return jnp.where(pred, mid + 1, lo), jnp.where(pred, hi, mid)

        lo, _ = lax.fori_loop(
            0, 14, bs_body,
            (jnp.zeros((16,), jnp.int32), jnp.full((16,), PAD, jnp.int32)),
        )
        e = jnp.minimum(lo, NFREQ - 1)
        left = plsc.load_gather(ybuf, [e])
        ebuf[...] = e
        ridx = plsc.load_gather(ebuf, [jnp.minimum(lanes + 1, 15)])
        ridx = jnp.where(lanes == 15, NFREQ - 1, ridx)
        right = plsc.load_gather(ybuf, [ridx])

        rs = _rsqrt_vec(fzero + total)
        spb = (right - left) * rs
        snr_splat = sum_y * rs
        d = spb - snr_splat * jnp.float32(1.0 / 16.0)
        chi = jnp.sum(d * d) * jnp.float32(16.0 / 15.0)
        return snr_splat, chi

    start_row(row0, h_a, s_a, sem_ha, sem_sa)

    def row_pair(g, carry):
        snr_res, chi_res = carry
        row_a = row0 + 2 * g
        row_b = row_a + 1
        start_row(row_b, h_b, s_b, sem_hb, sem_sb)
        wait_row(row_a, h_a, s_a, sem_ha, sem_sa)
        snr_v, chi_v = process_row(h_a, s_a)
        snr_res = jnp.where(lanes == 2 * g, snr_v, snr_res)
        chi_res = jnp.where(lanes == 2 * g, chi_v, chi_res)
        row_n = jnp.minimum(row_a + 2, row0 + ROWS_PER_W - 1)
        start_row(row_n, h_a, s_a, sem_ha, sem_sa)
        wait_row(row_b, h_b, s_b, sem_hb, sem_sb)
        snr_v, chi_v = process_row(h_b, s_b)
        snr_res = jnp.where(lanes == 2 * g + 1, snr_v, snr_res)
        chi_res = jnp.where(lanes == 2 * g + 1, chi_v, chi_res)
        return snr_res, chi_res

    snr_res, chi_res = lax.fori_loop(
        0, ROWS_PER_W // 2, row_pair, (fzero, fzero)
    )
    wait_row(row0 + ROWS_PER_W - 1, h_a, s_a, sem_ha, sem_sa)

    snrbuf[...] = snr_res
    chibuf[...] = chi_res
    pltpu.sync_copy(snrbuf, snr_hbm.at[pl.ds(row0, ROWS_PER_W)])
    pltpu.sync_copy(chibuf, chisq_hbm.at[pl.ds(row0, ROWS_PER_W)])


def kernel(htilde, stilde):
    b, c, f = htilde.shape
    snr, chisq = _chisq_sc(htilde.reshape(b * c, f), stilde.reshape(b * c, f))
    return snr.reshape(b, c), chisq.reshape(b, c)
